# baseline (speedup 1.0000x reference)
import functools

import jax
import jax.numpy as jnp
from jax import lax
from jax.experimental import pallas as pl
from jax.experimental.pallas import tpu as pltpu
from jax.experimental.pallas import tpu_sc as plsc

VOCAB = 1000000
EMBED = 64
HIDDEN = 128
NOUT = 2
BATCH = 4096
SEQ = 200

NC = 2
NS = 16
NW = NC * NS
B_PER_W = BATCH // NW
CHUNK = 100
CHUNKS_PER_ITEM = SEQ // CHUNK
N_CHUNKS = B_PER_W * CHUNKS_PER_ITEM
NBUF = 4
N_OUTER = N_CHUNKS // NBUF
NLANE = 16
NVEC = EMBED // NLANE


def _sc_gather_sum(xr, table):
    mesh = plsc.VectorSubcoreMesh(core_axis_name="c", subcore_axis_name="s")

    @functools.partial(
        pl.kernel,
        mesh=mesh,
        out_type=jax.ShapeDtypeStruct((BATCH, EMBED), jnp.float32),
        compiler_params=pltpu.CompilerParams(use_tc_tiling_on_sc=False),
        scratch_types=(
            [pltpu.VMEM((N_CHUNKS, CHUNK), jnp.int32),
             pltpu.VMEM((B_PER_W, EMBED), jnp.float32)]
            + [pltpu.VMEM((CHUNK, EMBED), jnp.float32) for _ in range(NBUF)]
            + [pltpu.SemaphoreType.DMA for _ in range(NBUF)]
        ),
    )
    def k(x_hbm, table_hbm, out_hbm, idx_v, out_v, *rest):
        bufs = rest[:NBUF]
        sems = rest[NBUF:]
        wid = lax.axis_index("s") * NC + lax.axis_index("c")
        ibase = wid * N_CHUNKS
        obase = wid * B_PER_W

        pltpu.sync_copy(x_hbm.at[pl.ds(ibase, N_CHUNKS)], idx_v)

        for b in range(NBUF):
            pltpu.async_copy(table_hbm.at[idx_v.at[b]], bufs[b], sems[b])

        def accum(buf, accs):
            def body(s, a):
                return tuple(
                    a[c] + buf[s, pl.ds(c * NLANE, NLANE)] for c in range(NVEC)
                )
            return lax.fori_loop(0, CHUNK, body, accs)

        def outer(t, carry):
            for pair in range(NBUF // CHUNKS_PER_ITEM):
                accs = tuple(
                    jnp.zeros((NLANE,), jnp.float32) for _ in range(NVEC)
                )
                for half in range(CHUNKS_PER_ITEM):
                    b = pair * CHUNKS_PER_ITEM + half
                    chunk = t * NBUF + b
                    pltpu.make_async_copy(
                        table_hbm.at[idx_v.at[chunk]], bufs[b], sems[b]
                    ).wait()
                    accs = accum(bufs[b], accs)

                    @pl.when(t < N_OUTER - 1)
                    def _fire():
                        pltpu.async_copy(
                            table_hbm.at[idx_v.at[chunk + NBUF]], bufs[b], sems[b]
                        )

                item = t * (NBUF // CHUNKS_PER_ITEM) + pair
                for c in range(NVEC):
                    out_v[item, pl.ds(c * NLANE, NLANE)] = accs[c]
            return carry

        lax.fori_loop(0, N_OUTER, outer, 0)
        pltpu.sync_copy(out_v, out_hbm.at[pl.ds(obase, B_PER_W)])

    return k(xr, table)


BM = 512
NOUT_PAD = 128


def _mlp_body(s_ref, w1_ref, b1_ref, w2_ref, b2_ref, o_ref):
    h = jnp.dot(s_ref[...] * (1.0 / SEQ), w1_ref[...],
                preferred_element_type=jnp.float32)
    h = jnp.maximum(h + b1_ref[...], 0.0)
    o_ref[...] = jnp.dot(h, w2_ref[...],
                         preferred_element_type=jnp.float32) + b2_ref[...]


def kernel(x, table, W1, b1, W2, b2):
    xr = x.astype(jnp.int32).reshape(BATCH * SEQ // CHUNK, CHUNK)
    sums = _sc_gather_sum(xr, table)

    w2p = jnp.zeros((HIDDEN, NOUT_PAD), W2.dtype).at[:, :NOUT].set(W2)
    b2p = jnp.zeros((1, NOUT_PAD), b2.dtype).at[0, :NOUT].set(b2)
    b1r = b1.reshape(1, HIDDEN)

    out = pl.pallas_call(
        _mlp_body,
        grid=(BATCH // BM,),
        in_specs=[
            pl.BlockSpec((BM, EMBED), lambda i: (i, 0)),
            pl.BlockSpec((EMBED, HIDDEN), lambda i: (0, 0)),
            pl.BlockSpec((1, HIDDEN), lambda i: (0, 0)),
            pl.BlockSpec((HIDDEN, NOUT_PAD), lambda i: (0, 0)),
            pl.BlockSpec((1, NOUT_PAD), lambda i: (0, 0)),
        ],
        out_specs=pl.BlockSpec((BM, NOUT_PAD), lambda i: (i, 0)),
        out_shape=jax.ShapeDtypeStruct((BATCH, NOUT_PAD), jnp.float32),
    )(sums, W1, b1r, w2p, b2p)
    return out[:, :NOUT]
